# Initial kernel scaffold; baseline (speedup 1.0000x reference)
import jax
import jax.numpy as jnp
from jax.experimental import pallas as pl


def kernel(feed_id, feed_emb, w2v_emb, author_emb, tag_emb, side_author, side_tags, W1, b1, W2, b2):
    raise NotImplementedError("write your pallas kernel here")


# speedup vs baseline: 8.3931x; 8.3931x over previous
import functools
import math

import jax
import jax.numpy as jnp
from jax import lax
from jax.experimental import pallas as pl
from jax.experimental.pallas import tpu as pltpu
from jax.experimental.pallas import tpu_sc as plsc

B = 16384
D = 128
VT = 1000
LT = 50


@functools.lru_cache(maxsize=None)
def _make_sc_gather(VF, VA, n):
  info = plsc.get_sparse_core_info()
  NC, NS = info.num_cores, info.num_subcores
  NW = NC * NS
  b_per_w = B // NW
  n_chunks = b_per_w // n
  mesh = plsc.VectorSubcoreMesh(core_axis_name="c", subcore_axis_name="s")

  @functools.partial(
      pl.kernel,
      out_type=(
          jax.ShapeDtypeStruct((B, D), jnp.float32),
          jax.ShapeDtypeStruct((B, D), jnp.float32),
          jax.ShapeDtypeStruct((B, D), jnp.float32),
          jax.ShapeDtypeStruct((B, D), jnp.int32),
      ),
      mesh=mesh,
      compiler_params=pltpu.CompilerParams(use_tc_tiling_on_sc=False),
      scratch_types=[
          pltpu.VMEM((n,), jnp.int32),
          pltpu.VMEM((n, D), jnp.float32),
          pltpu.VMEM((n, D), jnp.float32),
          pltpu.VMEM((n,), jnp.int32),
          pltpu.VMEM((n, D), jnp.float32),
          pltpu.VMEM((n, D), jnp.int32),
          pltpu.SemaphoreType.DMA,
          pltpu.SemaphoreType.DMA,
          pltpu.SemaphoreType.DMA,
          pltpu.SemaphoreType.DMA,
          pltpu.SemaphoreType.DMA,
      ],
  )
  def sc_gather(feed_id, feed_tab, w2v_tab, author_tab, sauthor, stags,
                e1_o, e2_o, a_o, t_o,
                idx_v, r1, r2, ai_v, ar, tg, s1, s2, s3, s4, s5):
    wid = lax.axis_index("s") * NC + lax.axis_index("c")
    base = wid * b_per_w

    def body(j, carry):
      off = base + j * n
      pltpu.sync_copy(feed_id.at[pl.ds(off, n)], idx_v)
      c1 = pltpu.async_copy(feed_tab.at[idx_v], r1, s1)
      c2 = pltpu.async_copy(w2v_tab.at[idx_v], r2, s2)
      c3 = pltpu.async_copy(sauthor.at[idx_v], ai_v, s3)
      c4 = pltpu.async_copy(stags.at[idx_v], tg, s4)
      c3.wait()
      c5 = pltpu.async_copy(author_tab.at[ai_v], ar, s5)
      c1.wait()
      pltpu.sync_copy(r1, e1_o.at[pl.ds(off, n)])
      c2.wait()
      pltpu.sync_copy(r2, e2_o.at[pl.ds(off, n)])
      c4.wait()
      pltpu.sync_copy(tg, t_o.at[pl.ds(off, n)])
      c5.wait()
      pltpu.sync_copy(ar, a_o.at[pl.ds(off, n)])
      return carry

    lax.fori_loop(0, n_chunks, body, 0)

  return sc_gather


def _tc_body(e1, e2, ao, ti, w1a, w1b, b1, w2a, w2b, w2c, b2, temb, out_ref):
  fe = (lax.dot_general(e1[...], w1a[...], (((1,), (0,)), ((), ())),
                        preferred_element_type=jnp.float32)
        + lax.dot_general(e2[...], w1b[...], (((1,), (0,)), ((), ())),
                          preferred_element_type=jnp.float32)
        + b1[...])

  S = lax.dot_general(fe, temb[...], (((1,), (1,)), ((), ())),
                      preferred_element_type=jnp.float32)
  S = S * jnp.float32(1.0 / math.sqrt(D))

  ti_b = ti[...][:, :LT]
  bB = ti_b.shape[0]
  iota_v = lax.broadcasted_iota(jnp.int32, (bB, VT), 1)
  c = jnp.zeros((bB, VT), jnp.float32)
  for l in range(LT):
    c = c + (ti_b[:, l:l + 1] == iota_v).astype(jnp.float32)

  S_masked = jnp.where(iota_v > 0, S, jnp.float32(-1e30))
  m = jnp.max(S_masked, axis=1, keepdims=True)
  E = c * jnp.exp(S_masked - m)
  Z = jnp.sum(E, axis=1, keepdims=True)
  good = Z > 0
  E = jnp.where(good, E, c)
  Z = jnp.where(good, Z, jnp.float32(LT))

  att = lax.dot_general(E, temb[...], (((1,), (0,)), ((), ())),
                        preferred_element_type=jnp.float32) / Z

  out = (lax.dot_general(fe, w2a[...], (((1,), (0,)), ((), ())),
                         preferred_element_type=jnp.float32)
         + lax.dot_general(ao[...], w2b[...], (((1,), (0,)), ((), ())),
                           preferred_element_type=jnp.float32)
         + lax.dot_general(att, w2c[...], (((1,), (0,)), ((), ())),
                           preferred_element_type=jnp.float32)
         + b2[...])
  out_ref[...] = out


@functools.lru_cache(maxsize=None)
def _make_tc(bB):
  grid = (B // bB,)
  row_spec = pl.BlockSpec((bB, D), lambda i: (i, 0))
  tag_spec = pl.BlockSpec((bB, D), lambda i: (i, 0))
  full = lambda shape: pl.BlockSpec(shape, lambda i: (0,) * len(shape))
  return pl.pallas_call(
      _tc_body,
      grid=grid,
      in_specs=[
          row_spec, row_spec, row_spec, tag_spec,
          full((D, D)), full((D, D)), full((1, D)),
          full((D, D)), full((D, D)), full((D, D)), full((1, D)),
          full((VT, D)),
      ],
      out_specs=row_spec,
      out_shape=jax.ShapeDtypeStruct((B, D), jnp.float32),
      compiler_params=pltpu.CompilerParams(
          dimension_semantics=("arbitrary",),
      ),
  )


def kernel(feed_id, feed_emb, w2v_emb, author_emb, tag_emb, side_author,
           side_tags, W1, b1, W2, b2):
  VF = feed_emb.shape[0]
  VA = author_emb.shape[0]
  sa_flat = side_author.reshape((VF,)).astype(jnp.int32)
  fid = feed_id.astype(jnp.int32)
  st_pad = jnp.pad(side_tags.astype(jnp.int32), ((0, 0), (0, D - LT)))

  sc = _make_sc_gather(VF, VA, 128)
  e1, e2, a_out, t_idx = sc(fid, feed_emb, w2v_emb, author_emb, sa_flat,
                            st_pad)

  tc = _make_tc(256)
  out = tc(e1, e2, a_out, t_idx,
           W1[:D], W1[D:], b1.reshape((1, D)),
           W2[:D], W2[D:2 * D], W2[2 * D:], b2.reshape((1, D)),
           tag_emb)
  return out


# speedup vs baseline: 8.4611x; 1.0081x over previous
import functools
import math

import jax
import jax.numpy as jnp
from jax import lax
from jax.experimental import pallas as pl
from jax.experimental.pallas import tpu as pltpu
from jax.experimental.pallas import tpu_sc as plsc

B = 16384
D = 128
VT = 1000
LT = 50


@functools.lru_cache(maxsize=None)
def _make_sc_gather(VF, VA, n, Bb):
  info = plsc.get_sparse_core_info()
  NC, NS = info.num_cores, info.num_subcores
  NW = NC * NS
  b_per_w = Bb // NW
  n_chunks = b_per_w // n
  mesh = plsc.VectorSubcoreMesh(core_axis_name="c", subcore_axis_name="s")

  @functools.partial(
      pl.kernel,
      out_type=(
          jax.ShapeDtypeStruct((Bb, D), jnp.float32),
          jax.ShapeDtypeStruct((Bb, D), jnp.float32),
          jax.ShapeDtypeStruct((Bb, D), jnp.float32),
          jax.ShapeDtypeStruct((Bb, D), jnp.int32),
      ),
      mesh=mesh,
      compiler_params=pltpu.CompilerParams(use_tc_tiling_on_sc=False),
      scratch_types=[
          pltpu.VMEM((n,), jnp.int32),
          pltpu.VMEM((n, D), jnp.float32),
          pltpu.VMEM((n, D), jnp.float32),
          pltpu.VMEM((n,), jnp.int32),
          pltpu.VMEM((n, D), jnp.float32),
          pltpu.VMEM((n, D), jnp.int32),
          pltpu.SemaphoreType.DMA,
          pltpu.SemaphoreType.DMA,
          pltpu.SemaphoreType.DMA,
          pltpu.SemaphoreType.DMA,
          pltpu.SemaphoreType.DMA,
      ],
  )
  def sc_gather(feed_id, feed_tab, w2v_tab, author_tab, sauthor, stags,
                e1_o, e2_o, a_o, t_o,
                idx_v, r1, r2, ai_v, ar, tg, s1, s2, s3, s4, s5):
    wid = lax.axis_index("s") * NC + lax.axis_index("c")
    base = wid * b_per_w

    def body(j, carry):
      off = base + j * n
      pltpu.sync_copy(feed_id.at[pl.ds(off, n)], idx_v)
      c1 = pltpu.async_copy(feed_tab.at[idx_v], r1, s1)
      c2 = pltpu.async_copy(w2v_tab.at[idx_v], r2, s2)
      c3 = pltpu.async_copy(sauthor.at[idx_v], ai_v, s3)
      c4 = pltpu.async_copy(stags.at[idx_v], tg, s4)
      c3.wait()
      c5 = pltpu.async_copy(author_tab.at[ai_v], ar, s5)
      c1.wait()
      pltpu.sync_copy(r1, e1_o.at[pl.ds(off, n)])
      c2.wait()
      pltpu.sync_copy(r2, e2_o.at[pl.ds(off, n)])
      c4.wait()
      pltpu.sync_copy(tg, t_o.at[pl.ds(off, n)])
      c5.wait()
      pltpu.sync_copy(ar, a_o.at[pl.ds(off, n)])
      return carry

    lax.fori_loop(0, n_chunks, body, 0)

  return sc_gather


def _tc_body(e1, e2, ao, ti, w1a, w1b, b1, w2a, w2b, w2c, b2, temb, out_ref):
  fe = (lax.dot_general(e1[...], w1a[...], (((1,), (0,)), ((), ())),
                        preferred_element_type=jnp.float32)
        + lax.dot_general(e2[...], w1b[...], (((1,), (0,)), ((), ())),
                          preferred_element_type=jnp.float32)
        + b1[...])

  S = lax.dot_general(fe, temb[...], (((1,), (1,)), ((), ())),
                      preferred_element_type=jnp.float32)
  S = S * jnp.float32(1.0 / math.sqrt(D))

  ti_b = ti[...][:, :LT]
  bB = ti_b.shape[0]
  iota_v = lax.broadcasted_iota(jnp.int32, (bB, VT), 1)
  c = jnp.zeros((bB, VT), jnp.float32)
  for l in range(LT):
    c = c + (ti_b[:, l:l + 1] == iota_v).astype(jnp.float32)

  S_masked = jnp.where(iota_v > 0, S, jnp.float32(-1e30))
  m = jnp.max(S_masked, axis=1, keepdims=True)
  E = c * jnp.exp(S_masked - m)
  Z = jnp.sum(E, axis=1, keepdims=True)
  good = Z > 0
  E = jnp.where(good, E, c)
  Z = jnp.where(good, Z, jnp.float32(LT))

  att = lax.dot_general(E, temb[...], (((1,), (0,)), ((), ())),
                        preferred_element_type=jnp.float32) / Z

  out = (lax.dot_general(fe, w2a[...], (((1,), (0,)), ((), ())),
                         preferred_element_type=jnp.float32)
         + lax.dot_general(ao[...], w2b[...], (((1,), (0,)), ((), ())),
                           preferred_element_type=jnp.float32)
         + lax.dot_general(att, w2c[...], (((1,), (0,)), ((), ())),
                           preferred_element_type=jnp.float32)
         + b2[...])
  out_ref[...] = out


@functools.lru_cache(maxsize=None)
def _make_tc(bB, Bb):
  grid = (Bb // bB,)
  row_spec = pl.BlockSpec((bB, D), lambda i: (i, 0))
  tag_spec = pl.BlockSpec((bB, D), lambda i: (i, 0))
  full = lambda shape: pl.BlockSpec(shape, lambda i: (0,) * len(shape))
  return pl.pallas_call(
      _tc_body,
      grid=grid,
      in_specs=[
          row_spec, row_spec, row_spec, tag_spec,
          full((D, D)), full((D, D)), full((1, D)),
          full((D, D)), full((D, D)), full((D, D)), full((1, D)),
          full((VT, D)),
      ],
      out_specs=row_spec,
      out_shape=jax.ShapeDtypeStruct((Bb, D), jnp.float32),
      compiler_params=pltpu.CompilerParams(
          dimension_semantics=("arbitrary",),
      ),
  )


def kernel(feed_id, feed_emb, w2v_emb, author_emb, tag_emb, side_author,
           side_tags, W1, b1, W2, b2):
  VF = feed_emb.shape[0]
  VA = author_emb.shape[0]
  sa_flat = side_author.reshape((VF,)).astype(jnp.int32)
  fid = feed_id.astype(jnp.int32)
  st_pad = jnp.pad(side_tags.astype(jnp.int32), ((0, 0), (0, D - LT)))

  n_split = 2
  Bb = B // n_split
  sc = _make_sc_gather(VF, VA, 128, Bb)
  tc = _make_tc(256, Bb)
  w_args = (W1[:D], W1[D:], b1.reshape((1, D)),
            W2[:D], W2[D:2 * D], W2[2 * D:], b2.reshape((1, D)), tag_emb)

  gathered = [sc(fid[i * Bb:(i + 1) * Bb], feed_emb, w2v_emb, author_emb,
                 sa_flat, st_pad) for i in range(n_split)]
  outs = [tc(*g, *w_args) for g in gathered]
  return jnp.concatenate(outs, axis=0)


# speedup vs baseline: 13.0075x; 1.5373x over previous
import functools
import math

import jax
import jax.numpy as jnp
from jax import lax
from jax.experimental import pallas as pl
from jax.experimental.pallas import tpu as pltpu
from jax.experimental.pallas import tpu_sc as plsc

B = 16384
D = 128
VT = 1000
VTP = 1024
LT = 50


@functools.lru_cache(maxsize=None)
def _make_sc_gather(VF, VA, n, Bb):
  info = plsc.get_sparse_core_info()
  NC, NS = info.num_cores, info.num_subcores
  NW = NC * NS
  b_per_w = Bb // NW
  n_chunks = b_per_w // n
  mesh = plsc.VectorSubcoreMesh(core_axis_name="c", subcore_axis_name="s")

  iota16 = lambda: lax.iota(jnp.int32, 16)

  @functools.partial(
      pl.kernel,
      out_type=(
          jax.ShapeDtypeStruct((Bb, D), jnp.float32),
          jax.ShapeDtypeStruct((Bb, D), jnp.float32),
          jax.ShapeDtypeStruct((Bb, D), jnp.float32),
          jax.ShapeDtypeStruct((Bb, VTP), jnp.float32),
      ),
      mesh=mesh,
      compiler_params=pltpu.CompilerParams(use_tc_tiling_on_sc=False,
                                           needs_layout_passes=False),
      scratch_types=[
          pltpu.VMEM((n,), jnp.int32),
          pltpu.VMEM((n, D), jnp.float32),
          pltpu.VMEM((n, D), jnp.float32),
          pltpu.VMEM((n,), jnp.int32),
          pltpu.VMEM((n, D), jnp.float32),
          pltpu.VMEM((n, D), jnp.int32),
          pltpu.VMEM((n, VTP), jnp.float32),
          pltpu.SemaphoreType.DMA,
          pltpu.SemaphoreType.DMA,
          pltpu.SemaphoreType.DMA,
          pltpu.SemaphoreType.DMA,
          pltpu.SemaphoreType.DMA,
      ],
  )
  def sc_gather(feed_id, feed_tab, w2v_tab, author_tab, sauthor, stags, hzero,
                e1_o, e2_o, a_o, c_o,
                idx_v, r1, r2, ai_v, ar, tg, hist, s1, s2, s3, s4, s5):
    wid = lax.axis_index("s") * NC + lax.axis_index("c")
    base = wid * b_per_w

    pltpu.sync_copy(hzero, hist)

    ones16 = jnp.ones((16,), jnp.float32)
    zeros16 = jnp.zeros((16,), jnp.float32)

    def body(j, carry):
      off = base + j * n
      pltpu.sync_copy(feed_id.at[pl.ds(off, n)], idx_v)
      c1 = pltpu.async_copy(feed_tab.at[idx_v], r1, s1)
      c2 = pltpu.async_copy(w2v_tab.at[idx_v], r2, s2)
      c3 = pltpu.async_copy(sauthor.at[idx_v], ai_v, s3)
      c4 = pltpu.async_copy(stags.at[idx_v], tg, s4)
      c3.wait()
      c5 = pltpu.async_copy(author_tab.at[ai_v], ar, s5)
      c4.wait()
      touched = []
      for g in range(n // 16):
        rows = g * 16 + iota16()
        for l in range(LT):
          lsplat = jnp.full((16,), l, jnp.int32)
          t16 = plsc.load_gather(tg, [rows, lsplat])
          plsc.addupdate_scatter(hist, [rows, t16], ones16)
          touched.append((rows, t16))
      c1.wait()
      pltpu.sync_copy(r1, e1_o.at[pl.ds(off, n)])
      c2.wait()
      pltpu.sync_copy(r2, e2_o.at[pl.ds(off, n)])
      c5.wait()
      pltpu.sync_copy(ar, a_o.at[pl.ds(off, n)])
      pltpu.sync_copy(hist, c_o.at[pl.ds(off, n)])
      for rows, t16 in touched:
        plsc.store_scatter(hist, [rows, t16], zeros16)
      return carry

    lax.fori_loop(0, n_chunks, body, 0)

  return sc_gather


def _tc_body(e1, e2, ao, cin, w1a, w1b, b1, w2a, w2b, w2c, b2, temb, out_ref):
  fe = (lax.dot_general(e1[...], w1a[...], (((1,), (0,)), ((), ())),
                        preferred_element_type=jnp.float32)
        + lax.dot_general(e2[...], w1b[...], (((1,), (0,)), ((), ())),
                          preferred_element_type=jnp.float32)
        + b1[...])

  S = lax.dot_general(fe, temb[...], (((1,), (1,)), ((), ())),
                      preferred_element_type=jnp.float32)
  S = S * jnp.float32(1.0 / math.sqrt(D))

  c = cin[...][:, :VT]
  bB = c.shape[0]
  iota_v = lax.broadcasted_iota(jnp.int32, (bB, VT), 1)

  S_masked = jnp.where(iota_v > 0, S, jnp.float32(-1e30))
  m = jnp.max(S_masked, axis=1, keepdims=True)
  E = c * jnp.exp(S_masked - m)
  Z = jnp.sum(E, axis=1, keepdims=True)
  good = Z > 0
  E = jnp.where(good, E, c)
  Z = jnp.where(good, Z, jnp.float32(LT))

  att = lax.dot_general(E, temb[...], (((1,), (0,)), ((), ())),
                        preferred_element_type=jnp.float32) / Z

  out = (lax.dot_general(fe, w2a[...], (((1,), (0,)), ((), ())),
                         preferred_element_type=jnp.float32)
         + lax.dot_general(ao[...], w2b[...], (((1,), (0,)), ((), ())),
                           preferred_element_type=jnp.float32)
         + lax.dot_general(att, w2c[...], (((1,), (0,)), ((), ())),
                           preferred_element_type=jnp.float32)
         + b2[...])
  out_ref[...] = out


@functools.lru_cache(maxsize=None)
def _make_tc(bB, Bb):
  grid = (Bb // bB,)
  row_spec = pl.BlockSpec((bB, D), lambda i: (i, 0))
  hist_spec = pl.BlockSpec((bB, VTP), lambda i: (i, 0))
  full = lambda shape: pl.BlockSpec(shape, lambda i: (0,) * len(shape))
  return pl.pallas_call(
      _tc_body,
      grid=grid,
      in_specs=[
          row_spec, row_spec, row_spec, hist_spec,
          full((D, D)), full((D, D)), full((1, D)),
          full((D, D)), full((D, D)), full((D, D)), full((1, D)),
          full((VT, D)),
      ],
      out_specs=row_spec,
      out_shape=jax.ShapeDtypeStruct((Bb, D), jnp.float32),
      compiler_params=pltpu.CompilerParams(
          dimension_semantics=("arbitrary",),
      ),
  )


def kernel(feed_id, feed_emb, w2v_emb, author_emb, tag_emb, side_author,
           side_tags, W1, b1, W2, b2):
  VF = feed_emb.shape[0]
  VA = author_emb.shape[0]
  sa_flat = side_author.reshape((VF,)).astype(jnp.int32)
  fid = feed_id.astype(jnp.int32)
  st_pad = jnp.pad(side_tags.astype(jnp.int32), ((0, 0), (0, D - LT)))

  n = 64
  n_split = 2
  Bb = B // n_split
  hzero = jnp.zeros((n, VTP), jnp.float32)
  sc = _make_sc_gather(VF, VA, n, Bb)
  tc = _make_tc(256, Bb)
  w_args = (W1[:D], W1[D:], b1.reshape((1, D)),
            W2[:D], W2[D:2 * D], W2[2 * D:], b2.reshape((1, D)), tag_emb)

  gathered = [sc(fid[i * Bb:(i + 1) * Bb], feed_emb, w2v_emb, author_emb,
                 sa_flat, st_pad, hzero) for i in range(n_split)]
  outs = [tc(*g, *w_args) for g in gathered]
  return jnp.concatenate(outs, axis=0)


# speedup vs baseline: 13.0616x; 1.0042x over previous
import functools
import math

import jax
import jax.numpy as jnp
from jax import lax
from jax.experimental import pallas as pl
from jax.experimental.pallas import tpu as pltpu
from jax.experimental.pallas import tpu_sc as plsc

B = 16384
D = 128
VT = 1000
VTP = 1024
LT = 50


@functools.lru_cache(maxsize=None)
def _make_sc_gather(VF, VA, n, Bb):
  info = plsc.get_sparse_core_info()
  NC, NS = info.num_cores, info.num_subcores
  NW = NC * NS
  b_per_w = Bb // NW
  n_chunks = b_per_w // n
  mesh = plsc.VectorSubcoreMesh(core_axis_name="c", subcore_axis_name="s")

  iota16 = lambda: lax.iota(jnp.int32, 16)

  @functools.partial(
      pl.kernel,
      out_type=(
          jax.ShapeDtypeStruct((Bb, D), jnp.float32),
          jax.ShapeDtypeStruct((Bb, D), jnp.float32),
          jax.ShapeDtypeStruct((Bb, D), jnp.float32),
          jax.ShapeDtypeStruct((Bb, VTP), jnp.float32),
      ),
      mesh=mesh,
      compiler_params=pltpu.CompilerParams(use_tc_tiling_on_sc=False,
                                           needs_layout_passes=False),
      scratch_types=[
          pltpu.VMEM((n,), jnp.int32),
          pltpu.VMEM((n, D), jnp.float32),
          pltpu.VMEM((n, D), jnp.float32),
          pltpu.VMEM((n,), jnp.int32),
          pltpu.VMEM((n, D), jnp.float32),
          pltpu.VMEM((n, D), jnp.int32),
          pltpu.VMEM((n, VTP), jnp.float32),
          pltpu.SemaphoreType.DMA,
          pltpu.SemaphoreType.DMA,
          pltpu.SemaphoreType.DMA,
          pltpu.SemaphoreType.DMA,
          pltpu.SemaphoreType.DMA,
      ],
  )
  def sc_gather(feed_id, feed_tab, w2v_tab, author_tab, sauthor, stags, hzero,
                e1_o, e2_o, a_o, c_o,
                idx_v, r1, r2, ai_v, ar, tg, hist, s1, s2, s3, s4, s5):
    wid = lax.axis_index("s") * NC + lax.axis_index("c")
    base = wid * b_per_w

    pltpu.sync_copy(hzero, hist)

    ones16 = jnp.ones((16,), jnp.float32)
    zeros16 = jnp.zeros((16,), jnp.float32)

    def body(j, carry):
      off = base + j * n
      pltpu.sync_copy(feed_id.at[pl.ds(off, n)], idx_v)
      c1 = pltpu.async_copy(feed_tab.at[idx_v], r1, s1)
      c2 = pltpu.async_copy(w2v_tab.at[idx_v], r2, s2)
      c3 = pltpu.async_copy(sauthor.at[idx_v], ai_v, s3)
      c4 = pltpu.async_copy(stags.at[idx_v], tg, s4)
      c3.wait()
      c5 = pltpu.async_copy(author_tab.at[ai_v], ar, s5)
      c4.wait()
      touched = []
      for g in range(n // 16):
        rows = g * 16 + iota16()
        for l in range(LT):
          lsplat = jnp.full((16,), l, jnp.int32)
          t16 = plsc.load_gather(tg, [rows, lsplat])
          plsc.addupdate_scatter(hist, [rows, t16], ones16)
          touched.append((rows, t16))
      c1.wait()
      pltpu.sync_copy(r1, e1_o.at[pl.ds(off, n)])
      c2.wait()
      pltpu.sync_copy(r2, e2_o.at[pl.ds(off, n)])
      c5.wait()
      pltpu.sync_copy(ar, a_o.at[pl.ds(off, n)])
      pltpu.sync_copy(hist, c_o.at[pl.ds(off, n)])
      for rows, t16 in touched:
        plsc.store_scatter(hist, [rows, t16], zeros16)
      return carry

    lax.fori_loop(0, n_chunks, body, 0)

  return sc_gather


def _tc_body(e1, e2, ao, cin, w1a, w1b, b1, w2a, w2b, w2c, b2, temb, out_ref):
  fe = (lax.dot_general(e1[...], w1a[...], (((1,), (0,)), ((), ())),
                        preferred_element_type=jnp.float32)
        + lax.dot_general(e2[...], w1b[...], (((1,), (0,)), ((), ())),
                          preferred_element_type=jnp.float32)
        + b1[...])

  S = lax.dot_general(fe, temb[...], (((1,), (1,)), ((), ())),
                      preferred_element_type=jnp.float32)
  S = S * jnp.float32(1.0 / math.sqrt(D))

  c = cin[...][:, :VT]
  bB = c.shape[0]
  iota_v = lax.broadcasted_iota(jnp.int32, (bB, VT), 1)

  S_masked = jnp.where(iota_v > 0, S, jnp.float32(-1e30))
  m = jnp.max(S_masked, axis=1, keepdims=True)
  E = c * jnp.exp(S_masked - m)
  Z = jnp.sum(E, axis=1, keepdims=True)
  good = Z > 0
  E = jnp.where(good, E, c)
  Z = jnp.where(good, Z, jnp.float32(LT))

  att = lax.dot_general(E, temb[...], (((1,), (0,)), ((), ())),
                        preferred_element_type=jnp.float32) / Z

  out = (lax.dot_general(fe, w2a[...], (((1,), (0,)), ((), ())),
                         preferred_element_type=jnp.float32)
         + lax.dot_general(ao[...], w2b[...], (((1,), (0,)), ((), ())),
                           preferred_element_type=jnp.float32)
         + lax.dot_general(att, w2c[...], (((1,), (0,)), ((), ())),
                           preferred_element_type=jnp.float32)
         + b2[...])
  out_ref[...] = out


@functools.lru_cache(maxsize=None)
def _make_tc(bB, Bb):
  grid = (Bb // bB,)
  row_spec = pl.BlockSpec((bB, D), lambda i: (i, 0))
  hist_spec = pl.BlockSpec((bB, VTP), lambda i: (i, 0))
  full = lambda shape: pl.BlockSpec(shape, lambda i: (0,) * len(shape))
  return pl.pallas_call(
      _tc_body,
      grid=grid,
      in_specs=[
          row_spec, row_spec, row_spec, hist_spec,
          full((D, D)), full((D, D)), full((1, D)),
          full((D, D)), full((D, D)), full((D, D)), full((1, D)),
          full((VT, D)),
      ],
      out_specs=row_spec,
      out_shape=jax.ShapeDtypeStruct((Bb, D), jnp.float32),
      compiler_params=pltpu.CompilerParams(
          dimension_semantics=("arbitrary",),
      ),
  )


def kernel(feed_id, feed_emb, w2v_emb, author_emb, tag_emb, side_author,
           side_tags, W1, b1, W2, b2):
  VF = feed_emb.shape[0]
  VA = author_emb.shape[0]
  sa_flat = side_author.reshape((VF,)).astype(jnp.int32)
  fid = feed_id.astype(jnp.int32)
  st_pad = jnp.pad(side_tags.astype(jnp.int32), ((0, 0), (0, D - LT)))

  n = 64
  n_split = 1
  Bb = B // n_split
  hzero = jnp.zeros((n, VTP), jnp.float32)
  sc = _make_sc_gather(VF, VA, n, Bb)
  tc = _make_tc(256, Bb)
  w_args = (W1[:D], W1[D:], b1.reshape((1, D)),
            W2[:D], W2[D:2 * D], W2[2 * D:], b2.reshape((1, D)), tag_emb)

  gathered = [sc(fid[i * Bb:(i + 1) * Bb], feed_emb, w2v_emb, author_emb,
                 sa_flat, st_pad, hzero) for i in range(n_split)]
  outs = [tc(*g, *w_args) for g in gathered]
  return jnp.concatenate(outs, axis=0)


# speedup vs baseline: 16.2207x; 1.2419x over previous
import functools
import math

import jax
import jax.numpy as jnp
from jax import lax
from jax.experimental import pallas as pl
from jax.experimental.pallas import tpu as pltpu
from jax.experimental.pallas import tpu_sc as plsc

B = 16384
D = 128
VT = 1000
VTP = 1024
LT = 50


@functools.lru_cache(maxsize=None)
def _make_sc_gather(VF, VA, n, Bb):
  info = plsc.get_sparse_core_info()
  NC, NS = info.num_cores, info.num_subcores
  NW = NC * NS
  b_per_w = Bb // NW
  n_chunks = b_per_w // n
  mesh = plsc.VectorSubcoreMesh(core_axis_name="c", subcore_axis_name="s")

  iota16 = lambda: lax.iota(jnp.int32, 16)

  @functools.partial(
      pl.kernel,
      out_type=(
          jax.ShapeDtypeStruct((Bb, D), jnp.float32),
          jax.ShapeDtypeStruct((Bb, D), jnp.float32),
          jax.ShapeDtypeStruct((Bb, D), jnp.float32),
          jax.ShapeDtypeStruct((Bb, VTP), jnp.float32),
      ),
      mesh=mesh,
      compiler_params=pltpu.CompilerParams(use_tc_tiling_on_sc=True,
                                           needs_layout_passes=False),
      scratch_types=[
          pltpu.VMEM((n,), jnp.int32),
          pltpu.VMEM((n, D), jnp.float32),
          pltpu.VMEM((n, D), jnp.float32),
          pltpu.VMEM((n,), jnp.int32),
          pltpu.VMEM((n, D), jnp.float32),
          pltpu.VMEM((n, D), jnp.int32),
          pltpu.VMEM((n, VTP), jnp.float32),
          pltpu.SemaphoreType.DMA,
          pltpu.SemaphoreType.DMA,
          pltpu.SemaphoreType.DMA,
          pltpu.SemaphoreType.DMA,
          pltpu.SemaphoreType.DMA,
      ],
  )
  def sc_gather(feed_id, feed_tab, w2v_tab, author_tab, sauthor, stags, hzero,
                e1_o, e2_o, a_o, c_o,
                idx_v, r1, r2, ai_v, ar, tg, hist, s1, s2, s3, s4, s5):
    wid = lax.axis_index("s") * NC + lax.axis_index("c")
    base = wid * b_per_w

    pltpu.sync_copy(hzero, hist)

    ones16 = jnp.ones((16,), jnp.float32)
    zeros16 = jnp.zeros((16,), jnp.float32)

    def body(j, carry):
      off = base + j * n
      pltpu.sync_copy(feed_id.at[pl.ds(off, n)], idx_v)
      c1 = pltpu.async_copy(feed_tab.at[idx_v], r1, s1)
      c2 = pltpu.async_copy(w2v_tab.at[idx_v], r2, s2)
      c3 = pltpu.async_copy(sauthor.at[idx_v], ai_v, s3)
      c4 = pltpu.async_copy(stags.at[idx_v], tg, s4)
      c3.wait()
      c5 = pltpu.async_copy(author_tab.at[ai_v], ar, s5)
      c4.wait()
      touched = []
      for g in range(n // 16):
        rows = g * 16 + iota16()
        for l in range(LT):
          lsplat = jnp.full((16,), l, jnp.int32)
          t16 = plsc.load_gather(tg, [rows, lsplat])
          plsc.addupdate_scatter(hist, [rows, t16], ones16)
          touched.append((rows, t16))
      c1.wait()
      pltpu.sync_copy(r1, e1_o.at[pl.ds(off, n)])
      c2.wait()
      pltpu.sync_copy(r2, e2_o.at[pl.ds(off, n)])
      c5.wait()
      pltpu.sync_copy(ar, a_o.at[pl.ds(off, n)])
      pltpu.sync_copy(hist, c_o.at[pl.ds(off, n)])
      for rows, t16 in touched:
        plsc.store_scatter(hist, [rows, t16], zeros16)
      return carry

    lax.fori_loop(0, n_chunks, body, 0)

  return sc_gather


def _tc_body(e1, e2, ao, cin, w1a, w1b, b1, w2a, w2b, w2c, b2, temb, out_ref):
  fe = (lax.dot_general(e1[...], w1a[...], (((1,), (0,)), ((), ())),
                        preferred_element_type=jnp.float32)
        + lax.dot_general(e2[...], w1b[...], (((1,), (0,)), ((), ())),
                          preferred_element_type=jnp.float32)
        + b1[...])

  S = lax.dot_general(fe, temb[...], (((1,), (1,)), ((), ())),
                      preferred_element_type=jnp.float32)
  S = S * jnp.float32(1.0 / math.sqrt(D))

  c = cin[...][:, :VT]
  bB = c.shape[0]
  iota_v = lax.broadcasted_iota(jnp.int32, (bB, VT), 1)

  S_masked = jnp.where(iota_v > 0, S, jnp.float32(-1e30))
  m = jnp.max(S_masked, axis=1, keepdims=True)
  E = c * jnp.exp(S_masked - m)
  Z = jnp.sum(E, axis=1, keepdims=True)
  good = Z > 0
  E = jnp.where(good, E, c)
  Z = jnp.where(good, Z, jnp.float32(LT))

  att = lax.dot_general(E, temb[...], (((1,), (0,)), ((), ())),
                        preferred_element_type=jnp.float32) / Z

  out = (lax.dot_general(fe, w2a[...], (((1,), (0,)), ((), ())),
                         preferred_element_type=jnp.float32)
         + lax.dot_general(ao[...], w2b[...], (((1,), (0,)), ((), ())),
                           preferred_element_type=jnp.float32)
         + lax.dot_general(att, w2c[...], (((1,), (0,)), ((), ())),
                           preferred_element_type=jnp.float32)
         + b2[...])
  out_ref[...] = out


@functools.lru_cache(maxsize=None)
def _make_tc(bB, Bb):
  grid = (Bb // bB,)
  row_spec = pl.BlockSpec((bB, D), lambda i: (i, 0))
  hist_spec = pl.BlockSpec((bB, VTP), lambda i: (i, 0))
  full = lambda shape: pl.BlockSpec(shape, lambda i: (0,) * len(shape))
  return pl.pallas_call(
      _tc_body,
      grid=grid,
      in_specs=[
          row_spec, row_spec, row_spec, hist_spec,
          full((D, D)), full((D, D)), full((1, D)),
          full((D, D)), full((D, D)), full((D, D)), full((1, D)),
          full((VT, D)),
      ],
      out_specs=row_spec,
      out_shape=jax.ShapeDtypeStruct((Bb, D), jnp.float32),
      compiler_params=pltpu.CompilerParams(
          dimension_semantics=("arbitrary",),
      ),
  )


def kernel(feed_id, feed_emb, w2v_emb, author_emb, tag_emb, side_author,
           side_tags, W1, b1, W2, b2):
  VF = feed_emb.shape[0]
  VA = author_emb.shape[0]
  sa_flat = side_author.reshape((VF,)).astype(jnp.int32)
  fid = feed_id.astype(jnp.int32)
  st_pad = jnp.pad(side_tags.astype(jnp.int32), ((0, 0), (0, D - LT)))

  n = 64
  n_split = 1
  Bb = B // n_split
  hzero = jnp.zeros((n, VTP), jnp.float32)
  sc = _make_sc_gather(VF, VA, n, Bb)
  tc = _make_tc(256, Bb)
  w_args = (W1[:D], W1[D:], b1.reshape((1, D)),
            W2[:D], W2[D:2 * D], W2[2 * D:], b2.reshape((1, D)), tag_emb)

  gathered = [sc(fid[i * Bb:(i + 1) * Bb], feed_emb, w2v_emb, author_emb,
                 sa_flat, st_pad, hzero) for i in range(n_split)]
  outs = [tc(*g, *w_args) for g in gathered]
  return jnp.concatenate(outs, axis=0)
